# initial kernel scaffold (unmeasured)
import functools

import jax
import jax.numpy as jnp
from jax import lax
from jax.experimental import pallas as pl
from jax.experimental.pallas import tpu as pltpu

N_DEV = 4


def _matmul_body(x_ref, w_ref, y_ref, amax_ref):
    j = pl.program_id(0)
    t = jnp.dot(x_ref[...], w_ref[...], preferred_element_type=jnp.float32)
    t = jnp.maximum(t, 0.0)
    y_ref[...] = t
    m = jnp.max(t)

    @pl.when(j == 0)
    def _():
        amax_ref[...] = jnp.full((8, 128), m, jnp.float32)

    @pl.when(j > 0)
    def _():
        amax_ref[...] = jnp.maximum(amax_ref[...], m)


def _matmul_relu_amax(x, w):
    m_per, k = x.shape
    _, n = w.shape
    bn = 1024
    grid = (n // bn,)
    return pl.pallas_call(
        _matmul_body,
        grid=grid,
        in_specs=[
            pl.BlockSpec((m_per, k), lambda j: (0, 0)),
            pl.BlockSpec((k, bn), lambda j: (0, j)),
        ],
        out_specs=[
            pl.BlockSpec((m_per, bn), lambda j: (0, j)),
            pl.BlockSpec((8, 128), lambda j: (0, 0)),
        ],
        out_shape=[
            jax.ShapeDtypeStruct((m_per, n), jnp.float32),
            jax.ShapeDtypeStruct((8, 128), jnp.float32),
        ],
        compiler_params=pltpu.CompilerParams(
            dimension_semantics=("arbitrary",),
        ),
    )(x, w)


def _a2a_body(y_ref, amax_ref, out_ref, amax_all, send_sems, recv_sems,
              amax_send_sems, amax_recv_sems):
    my = lax.axis_index("i")
    m_per = y_ref.shape[0]
    n_per = out_ref.shape[1]

    barrier_sem = pltpu.get_barrier_semaphore()
    for o in range(1, N_DEV):
        pl.semaphore_signal(
            barrier_sem, inc=1,
            device_id=((my + o) % N_DEV,),
            device_id_type=pl.DeviceIdType.MESH,
        )
    pl.semaphore_wait(barrier_sem, N_DEV - 1)

    amax_all[pl.ds(my, 1)] = amax_ref[...].reshape(1, 8, 128)
    out_ref[pl.ds(my * m_per, m_per), :] = y_ref[:, pl.ds(my * n_per, n_per)]

    rdmas = []
    for o in range(1, N_DEV):
        p = (my + o) % N_DEV
        am = pltpu.make_async_remote_copy(
            src_ref=amax_all.at[pl.ds(my, 1)],
            dst_ref=amax_all.at[pl.ds(my, 1)],
            send_sem=amax_send_sems.at[o],
            recv_sem=amax_recv_sems.at[N_DEV - o],
            device_id=(p,),
            device_id_type=pl.DeviceIdType.MESH,
        )
        am.start()
        dm = pltpu.make_async_remote_copy(
            src_ref=y_ref.at[:, pl.ds(p * n_per, n_per)],
            dst_ref=out_ref.at[pl.ds(my * m_per, m_per), :],
            send_sem=send_sems.at[o],
            recv_sem=recv_sems.at[N_DEV - o],
            device_id=(p,),
            device_id_type=pl.DeviceIdType.MESH,
        )
        dm.start()
        rdmas.append((am, dm))

    for am, dm in rdmas:
        am.wait()
        dm.wait()

    g = jnp.max(amax_all[...])
    scale = g / 448.0
    inv = 448.0 / g
    v = out_ref[...]
    q = (v * inv).astype(jnp.float8_e4m3fn).astype(jnp.float32)
    out_ref[...] = q * scale


def _a2a_epilogue(y, amax):
    m_per, n = y.shape
    n_per = n // N_DEV
    m_tot = m_per * N_DEV
    return pl.pallas_call(
        _a2a_body,
        out_shape=jax.ShapeDtypeStruct((m_tot, n_per), jnp.float32),
        in_specs=[
            pl.BlockSpec(memory_space=pltpu.VMEM),
            pl.BlockSpec(memory_space=pltpu.VMEM),
        ],
        out_specs=pl.BlockSpec(memory_space=pltpu.VMEM),
        scratch_shapes=[
            pltpu.VMEM((N_DEV, 8, 128), jnp.float32),
            pltpu.SemaphoreType.DMA((N_DEV,)),
            pltpu.SemaphoreType.DMA((N_DEV,)),
            pltpu.SemaphoreType.DMA((N_DEV,)),
            pltpu.SemaphoreType.DMA((N_DEV,)),
        ],
        compiler_params=pltpu.CompilerParams(collective_id=0),
    )(y, amax)


def kernel(x, w_mat):
    y, amax = _matmul_relu_amax(x, w_mat)
    return _a2a_epilogue(y, amax)


# baseline (device time: 314462 ns/iter reference)
import functools

import jax
import jax.numpy as jnp
from jax import lax
from jax.experimental import pallas as pl
from jax.experimental.pallas import tpu as pltpu

N_DEV = 4


def _matmul_body(x_ref, w_ref, y_ref, amax_ref):
    j = pl.program_id(0)
    t = jnp.dot(x_ref[...], w_ref[...], preferred_element_type=jnp.float32)
    t = jnp.maximum(t, 0.0)
    y_ref[...] = t
    m = jnp.max(t)

    @pl.when(j == 0)
    def _():
        amax_ref[...] = jnp.full((8, 128), m, jnp.float32)

    @pl.when(j > 0)
    def _():
        amax_ref[...] = jnp.maximum(amax_ref[...], m)


def _matmul_relu_amax(x, w):
    m_per, k = x.shape
    _, n = w.shape
    bn = 1024
    grid = (n // bn,)
    return pl.pallas_call(
        _matmul_body,
        grid=grid,
        in_specs=[
            pl.BlockSpec((m_per, k), lambda j: (0, 0)),
            pl.BlockSpec((k, bn), lambda j: (0, j)),
        ],
        out_specs=[
            pl.BlockSpec((m_per, bn), lambda j: (0, j)),
            pl.BlockSpec((8, 128), lambda j: (0, 0)),
        ],
        out_shape=[
            jax.ShapeDtypeStruct((m_per, n), jnp.float32),
            jax.ShapeDtypeStruct((8, 128), jnp.float32),
        ],
        compiler_params=pltpu.CompilerParams(
            dimension_semantics=("arbitrary",),
            vmem_limit_bytes=64 * 1024 * 1024,
        ),
    )(x, w)


def _a2a_body(y_ref, amax_ref, out_ref, amax_all, send_sems, recv_sems,
              amax_send_sems, amax_recv_sems, copy_sem):
    my = lax.axis_index("i")
    m_per = y_ref.shape[0]
    n_per = out_ref.shape[1]

    barrier_sem = pltpu.get_barrier_semaphore()
    for o in range(1, N_DEV):
        pl.semaphore_signal(
            barrier_sem, inc=1,
            device_id=((my + o) % N_DEV,),
            device_id_type=pl.DeviceIdType.MESH,
        )
    pl.semaphore_wait(barrier_sem, N_DEV - 1)

    amax_all[pl.ds(my, 1)] = amax_ref[...].reshape(1, 8, 128)
    local_copy = pltpu.make_async_copy(
        y_ref.at[:, pl.ds(my * n_per, n_per)],
        out_ref.at[pl.ds(my * m_per, m_per), :],
        copy_sem,
    )
    local_copy.start()

    rdmas = []
    for o in range(1, N_DEV):
        p = (my + o) % N_DEV
        am = pltpu.make_async_remote_copy(
            src_ref=amax_all.at[pl.ds(my, 1)],
            dst_ref=amax_all.at[pl.ds(my, 1)],
            send_sem=amax_send_sems.at[o],
            recv_sem=amax_recv_sems.at[N_DEV - o],
            device_id=(p,),
            device_id_type=pl.DeviceIdType.MESH,
        )
        am.start()
        dm = pltpu.make_async_remote_copy(
            src_ref=y_ref.at[:, pl.ds(p * n_per, n_per)],
            dst_ref=out_ref.at[pl.ds(my * m_per, m_per), :],
            send_sem=send_sems.at[o],
            recv_sem=recv_sems.at[N_DEV - o],
            device_id=(p,),
            device_id_type=pl.DeviceIdType.MESH,
        )
        dm.start()
        rdmas.append((am, dm))

    local_copy.wait()
    for am, dm in rdmas:
        am.wait()
        dm.wait()

    g = jnp.max(amax_all[...])
    scale = g / 448.0
    inv = 448.0 / g
    bm = 256
    m_tot = out_ref.shape[0]

    def ep(b, carry):
        v = out_ref[pl.ds(b * bm, bm), :]
        q = (v * inv).astype(jnp.float8_e4m3fn).astype(jnp.float32)
        out_ref[pl.ds(b * bm, bm), :] = q * scale
        return carry

    lax.fori_loop(0, m_tot // bm, ep, 0)


def _a2a_epilogue(y, amax):
    m_per, n = y.shape
    n_per = n // N_DEV
    m_tot = m_per * N_DEV
    return pl.pallas_call(
        _a2a_body,
        out_shape=jax.ShapeDtypeStruct((m_tot, n_per), jnp.float32),
        in_specs=[
            pl.BlockSpec(memory_space=pl.ANY),
            pl.BlockSpec(memory_space=pltpu.VMEM),
        ],
        out_specs=pl.BlockSpec(memory_space=pltpu.VMEM),
        scratch_shapes=[
            pltpu.VMEM((N_DEV, 8, 128), jnp.float32),
            pltpu.SemaphoreType.DMA((N_DEV,)),
            pltpu.SemaphoreType.DMA((N_DEV,)),
            pltpu.SemaphoreType.DMA((N_DEV,)),
            pltpu.SemaphoreType.DMA((N_DEV,)),
            pltpu.SemaphoreType.DMA,
        ],
        compiler_params=pltpu.CompilerParams(
            collective_id=0,
            vmem_limit_bytes=100 * 1024 * 1024,
        ),
    )(y, amax)


def kernel(x, w_mat):
    y, amax = _matmul_relu_amax(x, w_mat)
    return _a2a_epilogue(y, amax)


# device time: 186825 ns/iter; 1.6832x vs baseline; 1.6832x over previous
import functools

import jax
import jax.numpy as jnp
from jax import lax
from jax.experimental import pallas as pl
from jax.experimental.pallas import tpu as pltpu

N_DEV = 4


def _matmul_body(x_ref, w_ref, y_ref, amax_ref):
    j = pl.program_id(0)
    t = jnp.dot(x_ref[...], w_ref[...], preferred_element_type=jnp.float32)
    t = jnp.maximum(t, 0.0)
    y_ref[...] = t
    m = jnp.max(t)

    @pl.when(j == 0)
    def _():
        amax_ref[...] = jnp.full((8, 128), m, jnp.float32)

    @pl.when(j > 0)
    def _():
        amax_ref[...] = jnp.maximum(amax_ref[...], m)


def _matmul_relu_amax(x, w):
    m_per, k = x.shape
    _, n = w.shape
    bn = 1024
    grid = (n // bn,)
    return pl.pallas_call(
        _matmul_body,
        grid=grid,
        in_specs=[
            pl.BlockSpec((m_per, k), lambda j: (0, 0)),
            pl.BlockSpec((k, bn), lambda j: (0, j)),
        ],
        out_specs=[
            pl.BlockSpec((m_per, bn), lambda j: (0, j)),
            pl.BlockSpec((8, 128), lambda j: (0, 0)),
        ],
        out_shape=[
            jax.ShapeDtypeStruct((m_per, n), jnp.float32),
            jax.ShapeDtypeStruct((8, 128), jnp.float32),
        ],
        compiler_params=pltpu.CompilerParams(
            dimension_semantics=("arbitrary",),
            vmem_limit_bytes=64 * 1024 * 1024,
        ),
    )(x, w)


def _a2a_body(y_ref, amax_ref, out_ref, amax_all, qsend, qrecv, stage,
              send_sems, recv_sems, amax_send_sems, amax_recv_sems, copy_sem):
    my = lax.axis_index("i")
    m_per = y_ref.shape[0]
    n_per = out_ref.shape[1]
    bm = 256

    barrier_sem = pltpu.get_barrier_semaphore()
    for o in range(1, N_DEV):
        pl.semaphore_signal(
            barrier_sem, inc=1,
            device_id=((my + o) % N_DEV,),
            device_id_type=pl.DeviceIdType.MESH,
        )
    pl.semaphore_wait(barrier_sem, N_DEV - 1)

    amax_all[pl.ds(my, 1)] = amax_ref[...].reshape(1, 8, 128)
    am_rdmas = []
    for o in range(1, N_DEV):
        p = (my + o) % N_DEV
        am = pltpu.make_async_remote_copy(
            src_ref=amax_all.at[pl.ds(my, 1)],
            dst_ref=amax_all.at[pl.ds(my, 1)],
            send_sem=amax_send_sems.at[o],
            recv_sem=amax_recv_sems.at[N_DEV - o],
            device_id=(p,),
            device_id_type=pl.DeviceIdType.MESH,
        )
        am.start()
        am_rdmas.append(am)
    for am in am_rdmas:
        am.wait()
    g = jnp.max(amax_all[...])
    scale = g / 448.0
    inv = 448.0 / g

    rdmas = []
    for o in range(1, N_DEV):
        p = (my + o) % N_DEV
        st = o % 2
        fetch = pltpu.make_async_copy(
            y_ref.at[:, pl.ds(p * n_per, n_per)],
            stage.at[st],
            copy_sem,
        )
        fetch.start()
        fetch.wait()
        for b in range(m_per // bm):
            qsend[o - 1, pl.ds(b * bm, bm), :] = (
                stage[st, pl.ds(b * bm, bm), :] * inv
            ).astype(jnp.float8_e4m3fn)
        dm = pltpu.make_async_remote_copy(
            src_ref=qsend.at[o - 1],
            dst_ref=qrecv.at[N_DEV - o - 1],
            send_sem=send_sems.at[o],
            recv_sem=recv_sems.at[N_DEV - o],
            device_id=(p,),
            device_id_type=pl.DeviceIdType.MESH,
        )
        dm.start()
        rdmas.append(dm)

    fetch = pltpu.make_async_copy(
        y_ref.at[:, pl.ds(my * n_per, n_per)],
        stage.at[0],
        copy_sem,
    )
    fetch.start()
    fetch.wait()
    for b in range(m_per // bm):
        q = (stage[0, pl.ds(b * bm, bm), :] * inv).astype(jnp.float8_e4m3fn)
        out_ref[pl.ds(my * m_per + b * bm, bm), :] = (
            q.astype(jnp.float32) * scale
        )

    for o, dm in zip(range(1, N_DEV), rdmas):
        dm.wait()
        s = N_DEV - o - 1
        p = (my + N_DEV - o) % N_DEV
        for b in range(m_per // bm):
            out_ref[pl.ds(p * m_per + b * bm, bm), :] = (
                qrecv[s, pl.ds(b * bm, bm), :].astype(jnp.float32) * scale
            )


def _a2a_epilogue(y, amax):
    m_per, n = y.shape
    n_per = n // N_DEV
    m_tot = m_per * N_DEV
    return pl.pallas_call(
        _a2a_body,
        out_shape=jax.ShapeDtypeStruct((m_tot, n_per), jnp.float32),
        in_specs=[
            pl.BlockSpec(memory_space=pl.ANY),
            pl.BlockSpec(memory_space=pltpu.VMEM),
        ],
        out_specs=pl.BlockSpec(memory_space=pltpu.VMEM),
        scratch_shapes=[
            pltpu.VMEM((N_DEV, 8, 128), jnp.float32),
            pltpu.VMEM((N_DEV - 1, m_per, n_per), jnp.float8_e4m3fn),
            pltpu.VMEM((N_DEV - 1, m_per, n_per), jnp.float8_e4m3fn),
            pltpu.VMEM((2, m_per, n_per), jnp.float32),
            pltpu.SemaphoreType.DMA((N_DEV,)),
            pltpu.SemaphoreType.DMA((N_DEV,)),
            pltpu.SemaphoreType.DMA((N_DEV,)),
            pltpu.SemaphoreType.DMA((N_DEV,)),
            pltpu.SemaphoreType.DMA,
        ],
        compiler_params=pltpu.CompilerParams(
            collective_id=0,
            vmem_limit_bytes=100 * 1024 * 1024,
        ),
    )(y, amax)


def kernel(x, w_mat):
    y, amax = _matmul_relu_amax(x, w_mat)
    return _a2a_epilogue(y, amax)


# device time: 163108 ns/iter; 1.9279x vs baseline; 1.1454x over previous
import jax
import jax.numpy as jnp
from jax import lax
from jax.experimental import pallas as pl
from jax.experimental.pallas import tpu as pltpu

N_DEV = 4
BN = 256
NB = 32
NREM = 24
FSLOTS = 12


def _body(x_ref, w_ref, out_ref, wstage, fbuf, recvbuf, ownbuf,
          amax_acc, amax_all, ostage, wsems, send_sems, recv_sems,
          a_send, a_recv, out_sems):
    my = lax.axis_index("i")
    j = pl.program_id(0)
    m_per = x_ref.shape[0]

    def wfetch(jj, slot):
        cc = (8 * (my + 1) + jj) % NB
        pltpu.make_async_copy(
            w_ref.at[:, pl.ds(cc * BN, BN)],
            wstage.at[slot],
            wsems.at[slot],
        ).start()

    @pl.when(j == 0)
    def _():
        barrier_sem = pltpu.get_barrier_semaphore()
        for o in range(1, N_DEV):
            pl.semaphore_signal(
                barrier_sem, inc=1,
                device_id=((my + o) % N_DEV,),
                device_id_type=pl.DeviceIdType.MESH,
            )
        pl.semaphore_wait(barrier_sem, N_DEV - 1)
        wfetch(0, 0)
        wfetch(1, 1)

    pltpu.make_async_copy(
        w_ref.at[:, pl.ds(0, BN)], wstage.at[j % 3], wsems.at[j % 3]
    ).wait()

    t = jnp.dot(x_ref[...], wstage[j % 3], preferred_element_type=jnp.float32)
    t = jnp.maximum(t, 0.0)
    m = jnp.max(t)

    @pl.when(j == 0)
    def _():
        amax_acc[...] = jnp.full((8, 128), m, jnp.float32)

    @pl.when(j > 0)
    def _():
        amax_acc[...] = jnp.maximum(amax_acc[...], m)

    th = t.astype(jnp.bfloat16)

    @pl.when(j < NREM)
    def _():
        jf = j % FSLOTS
        @pl.when(j >= FSLOTS)
        def _():
            pltpu.make_async_remote_copy(
                src_ref=fbuf.at[pl.ds(jf, 1)],
                dst_ref=fbuf.at[pl.ds(jf, 1)],
                send_sem=send_sems.at[j - FSLOTS],
                recv_sem=recv_sems.at[0],
                device_id=(my,),
                device_id_type=pl.DeviceIdType.MESH,
            ).wait_send()

        fbuf[pl.ds(jf, 1)] = th[None]
        u = 3 - j // 8
        d = (my + j // 8 + 1) % N_DEV
        s = j % 8
        pltpu.make_async_remote_copy(
            src_ref=fbuf.at[pl.ds(jf, 1)],
            dst_ref=recvbuf.at[pl.ds(u - 1, 1), :, pl.ds(s * BN, BN)],
            send_sem=send_sems.at[j],
            recv_sem=recv_sems.at[(u - 1) * 8 + s],
            device_id=(d,),
            device_id_type=pl.DeviceIdType.MESH,
        ).start()

    @pl.when(j >= NREM)
    def _():
        ownbuf[:, pl.ds((j - NREM) * BN, BN)] = th

    @pl.when(j + 2 < NB)
    def _():
        wfetch(j + 2, (j + 2) % 3)

    @pl.when(j == NB - 1)
    def _():
        amax_all[pl.ds(my, 1)] = amax_acc[...].reshape(1, 8, 128)
        ams = []
        for o in range(1, N_DEV):
            p = (my + o) % N_DEV
            am = pltpu.make_async_remote_copy(
                src_ref=amax_all.at[pl.ds(my, 1)],
                dst_ref=amax_all.at[pl.ds(my, 1)],
                send_sem=a_send.at[o],
                recv_sem=a_recv.at[N_DEV - o],
                device_id=(p,),
                device_id_type=pl.DeviceIdType.MESH,
            )
            am.start()
            ams.append(am)
        for am in ams:
            am.wait()
        g = jnp.max(amax_all[...])
        scale = g / 448.0
        inv = 448.0 / g

        pending = []

        def flush(read_chunk, rows_start, half):
            oslot = len(pending) % 2
            if len(pending) >= 2:
                pending[-2].wait()
            for r in range(2):
                v = read_chunk(r, half)
                q = (v * inv).astype(jnp.float8_e4m3fn)
                ostage[oslot, pl.ds(r * 512, 512), :] = (
                    q.astype(jnp.float32) * scale
                )
            wdma = pltpu.make_async_copy(
                ostage.at[oslot],
                out_ref.at[pl.ds(rows_start, m_per),
                           pl.ds(half * 1024, 1024)],
                out_sems.at[oslot],
            )
            wdma.start()
            pending.append(wdma)

        def own_chunk(r, h):
            return ownbuf[pl.ds(r * 512, 512), pl.ds(h * 1024, 1024)]

        for h in (0, 1):
            flush(own_chunk, my * m_per, h)

        for u in (3, 2, 1):
            src = (my + u) % N_DEV
            for s in range(8):
                pltpu.make_async_remote_copy(
                    src_ref=fbuf.at[pl.ds(0, 1)],
                    dst_ref=recvbuf.at[pl.ds(u - 1, 1), :, pl.ds(s * BN, BN)],
                    send_sem=send_sems.at[0],
                    recv_sem=recv_sems.at[(u - 1) * 8 + s],
                    device_id=(src,),
                    device_id_type=pl.DeviceIdType.MESH,
                ).wait_recv()

            def rchunk(r, h, uu=u):
                return recvbuf[uu - 1, pl.ds(r * 512, 512),
                               pl.ds(h * 1024, 1024)]

            for h in (0, 1):
                flush(rchunk, src * m_per, h)

        for c in range(NREM - FSLOTS, NREM):
            pltpu.make_async_remote_copy(
                src_ref=fbuf.at[pl.ds(c % FSLOTS, 1)],
                dst_ref=fbuf.at[pl.ds(c % FSLOTS, 1)],
                send_sem=send_sems.at[c],
                recv_sem=recv_sems.at[0],
                device_id=(my,),
                device_id_type=pl.DeviceIdType.MESH,
            ).wait_send()

        for wdma in pending[-2:]:
            wdma.wait()


def kernel(x, w_mat):
    m_per, k = x.shape
    _, n = w_mat.shape
    n_per = n // N_DEV
    m_tot = m_per * N_DEV
    return pl.pallas_call(
        _body,
        grid=(NB,),
        out_shape=jax.ShapeDtypeStruct((m_tot, n_per), jnp.float32),
        in_specs=[
            pl.BlockSpec((m_per, k), lambda j: (0, 0)),
            pl.BlockSpec(memory_space=pl.ANY),
        ],
        out_specs=pl.BlockSpec(memory_space=pl.ANY),
        scratch_shapes=[
            pltpu.VMEM((3, k, BN), jnp.float32),
            pltpu.VMEM((FSLOTS, m_per, BN), jnp.bfloat16),
            pltpu.VMEM((3, m_per, n_per), jnp.bfloat16),
            pltpu.VMEM((m_per, n_per), jnp.bfloat16),
            pltpu.VMEM((8, 128), jnp.float32),
            pltpu.VMEM((N_DEV, 8, 128), jnp.float32),
            pltpu.VMEM((2, m_per, 1024), jnp.float32),
            pltpu.SemaphoreType.DMA((3,)),
            pltpu.SemaphoreType.DMA((NREM,)),
            pltpu.SemaphoreType.DMA((NREM,)),
            pltpu.SemaphoreType.DMA((N_DEV,)),
            pltpu.SemaphoreType.DMA((N_DEV,)),
            pltpu.SemaphoreType.DMA((2,)),
        ],
        compiler_params=pltpu.CompilerParams(
            collective_id=0,
            dimension_semantics=("arbitrary",),
            vmem_limit_bytes=64 * 1024 * 1024,
        ),
    )(x, w_mat)
